# Initial kernel scaffold; baseline (speedup 1.0000x reference)
import jax
import jax.numpy as jnp
from jax.experimental import pallas as pl


def kernel(x, gate_w, gate_b, w1, b1, w2, b2):
    raise NotImplementedError("write your pallas kernel here")


# speedup vs baseline: 1.2616x; 1.2616x over previous
import functools

import jax
import jax.numpy as jnp
from jax.experimental import pallas as pl
from jax.experimental.pallas import tpu as pltpu

N = 2048
D = 1024
E = 8
TOPK = 2
DFF = 4096
BT = 256
FBLK = 1024
NF = DFF // FBLK
NB = (N * TOPK) // BT + E


def _gate_kernel(x_ref, gw_ref, gb_ref, scores_ref, idx_ref, wt_ref):
    x = x_ref[...]
    gw = gw_ref[...]
    logits = jax.lax.dot_general(
        x, gw, (((1,), (1,)), ((), ())),
        preferred_element_type=jnp.float32) + gb_ref[...]
    scores = jax.nn.sigmoid(logits)
    scores_ref[...] = scores
    col = jax.lax.broadcasted_iota(jnp.int32, scores.shape, 1)
    m1 = jnp.max(scores, axis=1, keepdims=True)
    a1 = jnp.min(jnp.where(scores == m1, col, E), axis=1, keepdims=True)
    masked = jnp.where(col == a1, -1.0, scores)
    m2 = jnp.max(masked, axis=1, keepdims=True)
    a2 = jnp.min(jnp.where(masked == m2, col, E), axis=1, keepdims=True)
    z = jnp.zeros((x.shape[0], E - TOPK), dtype=jnp.int32)
    idx_ref[...] = jnp.concatenate([a1, a2, z], axis=1)
    wt_ref[...] = jnp.concatenate([m1, m2, z.astype(jnp.float32)], axis=1)


def _moe_kernel(nvb_ref, tok_ref, be_ref,
                x_ref, w1_ref, b1_ref, w2_ref, b2_ref, wgt_ref,
                out_ref, xs_ref, acc_ref):
    b = pl.program_id(0)
    f = pl.program_id(1)
    nv = nvb_ref[b]

    @pl.when(jnp.logical_and(b == 0, f == 0))
    def _init():
        out_ref[...] = jnp.zeros_like(out_ref)

    @pl.when(jnp.logical_and(nv > 0, f == 0))
    def _gather():
        def body(r, _):
            t = tok_ref[b, r]
            xs_ref[pl.ds(r, 1), :] = x_ref[pl.ds(t, 1), :]
            return 0
        jax.lax.fori_loop(0, BT, body, 0, unroll=True)

    @pl.when(nv > 0)
    def _compute():
        xs = xs_ref[...]
        h = jax.lax.dot_general(
            xs, w1_ref[0], (((1,), (1,)), ((), ())),
            preferred_element_type=jnp.float32) + b1_ref[0]
        h = jax.nn.gelu(h, approximate=True)
        part = jax.lax.dot_general(
            h, w2_ref[0], (((1,), (1,)), ((), ())),
            preferred_element_type=jnp.float32)

        @pl.when(f == 0)
        def _first():
            acc_ref[...] = part + b2_ref[0]

        @pl.when(f > 0)
        def _rest():
            acc_ref[...] += part

    @pl.when(jnp.logical_and(nv > 0, f == NF - 1))
    def _scatter():
        w = wgt_ref[0]
        acc_ref[...] *= w.reshape(BT, 1)
        def body(r, _):
            t = tok_ref[b, r]
            out_ref[pl.ds(t, 1), :] += acc_ref[pl.ds(r, 1), :]
            return 0
        jax.lax.fori_loop(0, nv, body, 0)


@functools.partial(jax.jit, static_argnames=())
def kernel(x, gate_w, gate_b, w1, b1, w2, b2):
    bsz, t, d = x.shape
    x_flat = x.reshape(N, D)

    scores, idx8, wt8 = pl.pallas_call(
        _gate_kernel,
        out_shape=(
            jax.ShapeDtypeStruct((N, E), jnp.float32),
            jax.ShapeDtypeStruct((N, E), jnp.int32),
            jax.ShapeDtypeStruct((N, E), jnp.float32),
        ),
    )(x_flat, gate_w, gate_b)

    ea = idx8[:, :TOPK].reshape(-1)
    wa = wt8[:, :TOPK].reshape(-1)
    ta = (jnp.arange(N * TOPK, dtype=jnp.int32) // TOPK)
    order = jnp.argsort(ea, stable=True)
    counts = jnp.sum(ea[None, :] == jnp.arange(E, dtype=jnp.int32)[:, None],
                     axis=1).astype(jnp.int32)
    starts = jnp.cumsum(counts) - counts
    nb_e = (counts + BT - 1) // BT
    cumnb = jnp.cumsum(nb_e)
    nb_total = cumnb[-1]
    bidx = jnp.arange(NB, dtype=jnp.int32)
    be = jnp.minimum(
        jnp.searchsorted(cumnb, bidx, side="right"), E - 1).astype(jnp.int32)
    block_rank = bidx - (cumnb - nb_e)[be]
    rr = jnp.arange(BT, dtype=jnp.int32)
    gpos = starts[be][:, None] + block_rank[:, None] * BT + rr[None, :]
    nvalid = jnp.where(bidx < nb_total,
                       jnp.clip(counts[be] - block_rank * BT, 0, BT),
                       0).astype(jnp.int32)
    valid = rr[None, :] < nvalid[:, None]
    aidx = order[jnp.clip(gpos, 0, N * TOPK - 1)]
    tok = jnp.where(valid, ta[aidx], 0).astype(jnp.int32)
    wgt = jnp.where(valid, wa[aidx], 0.0).reshape(NB, 1, BT)

    b1r = b1.reshape(E, 1, DFF)
    b2r = b2.reshape(E, 1, D)

    grid_spec = pltpu.PrefetchScalarGridSpec(
        num_scalar_prefetch=3,
        grid=(NB, NF),
        in_specs=[
            pl.BlockSpec((N, D), lambda b, f, nvb, tok, be: (0, 0)),
            pl.BlockSpec((1, FBLK, D), lambda b, f, nvb, tok, be: (be[b], f, 0)),
            pl.BlockSpec((1, 1, FBLK), lambda b, f, nvb, tok, be: (be[b], 0, f)),
            pl.BlockSpec((1, D, FBLK), lambda b, f, nvb, tok, be: (be[b], 0, f)),
            pl.BlockSpec((1, 1, D), lambda b, f, nvb, tok, be: (be[b], 0, 0)),
            pl.BlockSpec((1, 1, BT), lambda b, f, nvb, tok, be: (b, 0, 0)),
        ],
        out_specs=pl.BlockSpec((N, D), lambda b, f, nvb, tok, be: (0, 0)),
        scratch_shapes=[
            pltpu.VMEM((BT, D), jnp.float32),
            pltpu.VMEM((BT, D), jnp.float32),
        ],
    )

    out = pl.pallas_call(
        _moe_kernel,
        grid_spec=grid_spec,
        out_shape=jax.ShapeDtypeStruct((N, D), jnp.float32),
        compiler_params=pltpu.CompilerParams(
            dimension_semantics=("arbitrary", "arbitrary"),
        ),
    )(nvalid, tok, be, x_flat, w1, b1r, w2, b2r, wgt)

    return (out.reshape(bsz, t, d), scores.reshape(bsz, t, E))


# speedup vs baseline: 1.4977x; 1.1872x over previous
import jax
import jax.numpy as jnp
from jax.experimental import pallas as pl
from jax.experimental.pallas import tpu as pltpu

N = 2048
D = 1024
E = 8
TOPK = 2
DFF = 4096
BT = 256
FBLK = 1024
NF = DFF // FBLK
NBE = N // BT
NA = N * TOPK


def _gate_kernel(x_ref, gw_ref, gb_ref, scores_ref, idx_ref, wt_ref):
    x = x_ref[...]
    gw = gw_ref[...]
    logits = jax.lax.dot_general(
        x, gw, (((1,), (1,)), ((), ())),
        preferred_element_type=jnp.float32) + gb_ref[...]
    scores = jax.nn.sigmoid(logits)
    scores_ref[...] = scores
    col = jax.lax.broadcasted_iota(jnp.int32, scores.shape, 1)
    m1 = jnp.max(scores, axis=1, keepdims=True)
    a1 = jnp.min(jnp.where(scores == m1, col, E), axis=1, keepdims=True)
    masked = jnp.where(col == a1, -1.0, scores)
    m2 = jnp.max(masked, axis=1, keepdims=True)
    a2 = jnp.min(jnp.where(masked == m2, col, E), axis=1, keepdims=True)
    z = jnp.zeros((x.shape[0], E - TOPK), dtype=jnp.int32)
    idx_ref[...] = jnp.concatenate([a1, a2, z], axis=1)
    wt_ref[...] = jnp.concatenate([m1, m2, z.astype(jnp.float32)], axis=1)


def _moe_kernel(counts_ref, starts_ref, tok_ref,
                x_ref, w1_ref, b1_ref, w2_ref, b2_ref, wgt_ref,
                out_ref, xg_ref, acc_ref):
    e = pl.program_id(0)
    f = pl.program_id(1)
    b = pl.program_id(2)
    off = starts_ref[e] + b * BT
    nv = jnp.clip(counts_ref[e] - b * BT, 0, BT)
    row = pl.ds(b * BT, BT)

    @pl.when(jnp.logical_and(e == 0, jnp.logical_and(f == 0, b == 0)))
    def _init():
        out_ref[...] = jnp.zeros_like(out_ref)

    @pl.when(nv > 0)
    def _work():
        @pl.when(f == 0)
        def _gather():
            def body(r, _):
                t = tok_ref[off + r]
                xg_ref[pl.ds(b * BT + r, 1), :] = x_ref[pl.ds(t, 1), :]
                return 0
            jax.lax.fori_loop(0, BT, body, 0, unroll=True)

        xs = xg_ref[row, :]
        h = jax.lax.dot_general(
            xs, w1_ref[0], (((1,), (1,)), ((), ())),
            preferred_element_type=jnp.float32) + b1_ref[0]
        h = jax.nn.gelu(h, approximate=True)
        part = jax.lax.dot_general(
            h, w2_ref[0], (((1,), (1,)), ((), ())),
            preferred_element_type=jnp.float32)

        @pl.when(f == 0)
        def _first():
            acc_ref[row, :] = part + b2_ref[0]

        @pl.when(f > 0)
        def _rest():
            acc_ref[row, :] += part

        @pl.when(f == NF - 1)
        def _scatter():
            ridx = jax.lax.broadcasted_iota(jnp.int32, (BT, 1), 0)
            w = jnp.where(ridx < nv, wgt_ref[pl.ds(off, BT), :], 0.0)
            acc_ref[row, :] *= w
            def body(r, _):
                t = tok_ref[off + r]
                out_ref[pl.ds(t, 1), :] += acc_ref[pl.ds(b * BT + r, 1), :]
                return 0
            jax.lax.fori_loop(0, BT, body, 0, unroll=True)


def kernel(x, gate_w, gate_b, w1, b1, w2, b2):
    bsz, t, d = x.shape
    x_flat = x.reshape(N, D)

    scores, idx8, wt8 = pl.pallas_call(
        _gate_kernel,
        out_shape=(
            jax.ShapeDtypeStruct((N, E), jnp.float32),
            jax.ShapeDtypeStruct((N, E), jnp.int32),
            jax.ShapeDtypeStruct((N, E), jnp.float32),
        ),
    )(x_flat, gate_w, gate_b)

    ea = idx8[:, :TOPK].reshape(-1)
    wa = wt8[:, :TOPK].reshape(-1)
    ta = (jnp.arange(NA, dtype=jnp.int32) // TOPK)
    _, sorted_tok, sorted_w = jax.lax.sort(
        (ea, ta, wa), dimension=0, is_stable=True, num_keys=1)
    counts = jnp.sum(ea[None, :] == jnp.arange(E, dtype=jnp.int32)[:, None],
                     axis=1).astype(jnp.int32)
    starts = (jnp.cumsum(counts) - counts).astype(jnp.int32)
    tok_pad = jnp.concatenate(
        [sorted_tok, jnp.zeros((BT,), jnp.int32)]).astype(jnp.int32)
    wgt_pad = jnp.concatenate(
        [sorted_w, jnp.zeros((BT,), jnp.float32)]).reshape(NA + BT, 1)

    b1r = b1.reshape(E, 1, DFF)
    b2r = b2.reshape(E, 1, D)

    grid_spec = pltpu.PrefetchScalarGridSpec(
        num_scalar_prefetch=3,
        grid=(E, NF, NBE),
        in_specs=[
            pl.BlockSpec((N, D), lambda e, f, b, *s: (0, 0)),
            pl.BlockSpec((1, FBLK, D), lambda e, f, b, *s: (e, f, 0)),
            pl.BlockSpec((1, 1, FBLK), lambda e, f, b, *s: (e, 0, f)),
            pl.BlockSpec((1, D, FBLK), lambda e, f, b, *s: (e, 0, f)),
            pl.BlockSpec((1, 1, D), lambda e, f, b, *s: (e, 0, 0)),
            pl.BlockSpec((NA + BT, 1), lambda e, f, b, *s: (0, 0)),
        ],
        out_specs=pl.BlockSpec((N, D), lambda e, f, b, *s: (0, 0)),
        scratch_shapes=[
            pltpu.VMEM((N, D), jnp.float32),
            pltpu.VMEM((N, D), jnp.float32),
        ],
    )

    out = pl.pallas_call(
        _moe_kernel,
        grid_spec=grid_spec,
        out_shape=jax.ShapeDtypeStruct((N, D), jnp.float32),
        compiler_params=pltpu.CompilerParams(
            dimension_semantics=("arbitrary", "arbitrary", "arbitrary"),
        ),
    )(counts, starts, tok_pad, x_flat, w1, b1r, w2, b2r, wgt_pad)

    return (out.reshape(bsz, t, d), scores.reshape(bsz, t, E))


# speedup vs baseline: 2.2538x; 1.5048x over previous
import jax
import jax.numpy as jnp
from jax.experimental import pallas as pl
from jax.experimental.pallas import tpu as pltpu

N = 2048
D = 1024
E = 8
TOPK = 2
DFF = 4096
BT = 256
FBLK = 1024
NF = DFF // FBLK
NA = N * TOPK


def _gate_kernel(x_ref, gw_ref, gb_ref, scores_ref, idx_ref, wt_ref):
    x = x_ref[...]
    gw = gw_ref[...]
    logits = jax.lax.dot_general(
        x, gw, (((1,), (1,)), ((), ())),
        preferred_element_type=jnp.float32) + gb_ref[...]
    scores = jax.nn.sigmoid(logits)
    scores_ref[...] = scores
    col = jax.lax.broadcasted_iota(jnp.int32, scores.shape, 1)
    m1 = jnp.max(scores, axis=1, keepdims=True)
    a1 = jnp.min(jnp.where(scores == m1, col, E), axis=1, keepdims=True)
    masked = jnp.where(col == a1, -1.0, scores)
    m2 = jnp.max(masked, axis=1, keepdims=True)
    a2 = jnp.min(jnp.where(masked == m2, col, E), axis=1, keepdims=True)
    z = jnp.zeros((x.shape[0], E - TOPK), dtype=jnp.int32)
    idx_ref[...] = jnp.concatenate([a1, a2, z], axis=1)
    wt_ref[...] = jnp.concatenate([m1, m2, z.astype(jnp.float32)], axis=1)


def _moe_kernel(counts_ref, starts_ref, tok_ref,
                x_ref, w1_ref, b1_ref, w2_ref, b2_ref, wgt_ref,
                out_ref, xg_ref, acc_ref):
    e = pl.program_id(0)
    f = pl.program_id(1)
    cnt = counts_ref[e]
    start = starts_ref[e]
    nblk = (cnt + BT - 1) // BT

    @pl.when(jnp.logical_and(e == 0, f == 0))
    def _init():
        out_ref[...] = jnp.zeros_like(out_ref)

    def block_body(b, _):
        off = start + b * BT
        nv = jnp.clip(cnt - b * BT, 0, BT)
        row = pl.ds(b * BT, BT)

        @pl.when(f == 0)
        def _gather():
            def gbody(r, _):
                t = tok_ref[off + r]
                xg_ref[pl.ds(b * BT + r, 1), :] = x_ref[pl.ds(t, 1), :]
                return 0
            jax.lax.fori_loop(0, BT, gbody, 0, unroll=True)

        xs = xg_ref[row, :]
        h = jax.lax.dot_general(
            xs, w1_ref[0], (((1,), (1,)), ((), ())),
            preferred_element_type=jnp.float32) + b1_ref[0]
        h = jax.nn.gelu(h, approximate=True)
        part = jax.lax.dot_general(
            h, w2_ref[0], (((1,), (1,)), ((), ())),
            preferred_element_type=jnp.float32)

        @pl.when(f == 0)
        def _first():
            acc_ref[row, :] = part + b2_ref[0]

        @pl.when(f > 0)
        def _rest():
            acc_ref[row, :] += part

        @pl.when(f == NF - 1)
        def _scatter():
            ridx = jax.lax.broadcasted_iota(jnp.int32, (BT, 1), 0)
            w = jnp.where(ridx < nv, wgt_ref[pl.ds(off, BT), :], 0.0)
            acc_ref[row, :] *= w
            def sbody(r, _):
                t = tok_ref[off + r]
                out_ref[pl.ds(t, 1), :] += acc_ref[pl.ds(b * BT + r, 1), :]
                return 0
            jax.lax.fori_loop(0, BT, sbody, 0, unroll=True)
        return 0

    jax.lax.fori_loop(0, nblk, block_body, 0)


def kernel(x, gate_w, gate_b, w1, b1, w2, b2):
    bsz, t, d = x.shape
    x_flat = x.reshape(N, D)

    scores, idx8, wt8 = pl.pallas_call(
        _gate_kernel,
        out_shape=(
            jax.ShapeDtypeStruct((N, E), jnp.float32),
            jax.ShapeDtypeStruct((N, E), jnp.int32),
            jax.ShapeDtypeStruct((N, E), jnp.float32),
        ),
    )(x_flat, gate_w, gate_b)

    ea = idx8[:, :TOPK].reshape(-1)
    wa = wt8[:, :TOPK].reshape(-1)
    ta = (jnp.arange(NA, dtype=jnp.int32) // TOPK)
    _, sorted_tok, sorted_w = jax.lax.sort(
        (ea, ta, wa), dimension=0, is_stable=True, num_keys=1)
    counts = jnp.sum(ea[None, :] == jnp.arange(E, dtype=jnp.int32)[:, None],
                     axis=1).astype(jnp.int32)
    starts = (jnp.cumsum(counts) - counts).astype(jnp.int32)
    tok_pad = jnp.concatenate(
        [sorted_tok, jnp.zeros((BT,), jnp.int32)]).astype(jnp.int32)
    wgt_pad = jnp.concatenate(
        [sorted_w, jnp.zeros((BT,), jnp.float32)]).reshape(NA + BT, 1)

    b1r = b1.reshape(E, 1, DFF)
    b2r = b2.reshape(E, 1, D)

    grid_spec = pltpu.PrefetchScalarGridSpec(
        num_scalar_prefetch=3,
        grid=(E, NF),
        in_specs=[
            pl.BlockSpec((N, D), lambda e, f, *s: (0, 0)),
            pl.BlockSpec((1, FBLK, D), lambda e, f, *s: (e, f, 0)),
            pl.BlockSpec((1, 1, FBLK), lambda e, f, *s: (e, 0, f)),
            pl.BlockSpec((1, D, FBLK), lambda e, f, *s: (e, 0, f)),
            pl.BlockSpec((1, 1, D), lambda e, f, *s: (e, 0, 0)),
            pl.BlockSpec((NA + BT, 1), lambda e, f, *s: (0, 0)),
        ],
        out_specs=pl.BlockSpec((N, D), lambda e, f, *s: (0, 0)),
        scratch_shapes=[
            pltpu.VMEM((N, D), jnp.float32),
            pltpu.VMEM((N, D), jnp.float32),
        ],
    )

    out = pl.pallas_call(
        _moe_kernel,
        grid_spec=grid_spec,
        out_shape=jax.ShapeDtypeStruct((N, D), jnp.float32),
        compiler_params=pltpu.CompilerParams(
            dimension_semantics=("arbitrary", "arbitrary"),
        ),
    )(counts, starts, tok_pad, x_flat, w1, b1r, w2, b2r, wgt_pad)

    return (out.reshape(bsz, t, d), scores.reshape(bsz, t, E))
